# baseline (speedup 1.0000x reference)
import jax
import jax.numpy as jnp
from jax import lax
from jax.experimental import pallas as pl
from jax.experimental.pallas import tpu as pltpu
from jax.experimental.pallas import tpu_sc as plsc

N = 10000
E = 320000
C_IN = 128
C_OUT = 128
H = 64
NPAD = 10240
CH = 160
ECH = 64
EPW = ECH * CH
EPAD = 32 * EPW


def _group_mats(c, groups):
    return jnp.repeat(jnp.eye(groups, dtype=jnp.float32), c // groups, axis=0)


def _gn_elu(x, gamma, beta, gmat, eps=1e-5):
    per = gmat.sum(axis=0)
    mean = ((x @ gmat) / per) @ gmat.T
    d = x - mean
    var = (((d * d) @ gmat) / per) @ gmat.T
    y = d * lax.rsqrt(var + eps) * gamma + beta
    return jnp.where(y > 0, y, jnp.exp(jnp.minimum(y, 0.0)) - 1.0)


def _att_outs(h, atts, attd, h_ref, as_ref, ad_ref, m_ref):
    a_s = jnp.sum(h * atts, axis=1, keepdims=True)
    a_d = jnp.sum(h * attd, axis=1, keepdims=True)
    t = jnp.max(a_s) + a_d
    m = jnp.maximum(t, 0.2 * t)
    h_ref[...] = h
    as_ref[...] = a_s
    ad_ref[...] = a_d
    m_ref[...] = m


def _dense_pre_body(x_ref, lin1w_ref, lin1b_ref, convw_ref, atts_ref, attd_ref,
                    preg_ref, preb_ref, n1g_ref, n1b_ref,
                    h_ref, as_ref, ad_ref, m_ref):
    g16 = _group_mats(C_IN, C_IN // 8)
    g8 = _group_mats(H, H // 8)
    y = _gn_elu(x_ref[...], preg_ref[...], preb_ref[...], g16)
    y = jnp.dot(y, lin1w_ref[...].T, preferred_element_type=jnp.float32) + lin1b_ref[...]
    y = _gn_elu(y, n1g_ref[...], n1b_ref[...], g8)
    h = jnp.dot(y, convw_ref[...].T, preferred_element_type=jnp.float32)
    _att_outs(h, atts_ref[...], attd_ref[...], h_ref, as_ref, ad_ref, m_ref)


def _dense_mid_body(acc_ref, den_ref, bias_ref, convw_ref, atts_ref, attd_ref,
                    ng_ref, nb_ref, h_ref, as_ref, ad_ref, m_ref):
    g8 = _group_mats(H, H // 8)
    den = den_ref[0, :N] + den_ref[1, :N] + 1e-16
    z = (acc_ref[0, :N] + acc_ref[1, :N]) / den[:, None] + bias_ref[...]
    y = _gn_elu(z, ng_ref[...], nb_ref[...], g8)
    h = jnp.dot(y, convw_ref[...].T, preferred_element_type=jnp.float32)
    _att_outs(h, atts_ref[...], attd_ref[...], h_ref, as_ref, ad_ref, m_ref)


def _dense_post_body(x_ref, acc_ref, den_ref, bias_ref, ng_ref, nb_ref,
                     lin2w_ref, lin2b_ref, out_ref):
    g8 = _group_mats(H, H // 8)
    den = den_ref[0, :N] + den_ref[1, :N] + 1e-16
    z = (acc_ref[0, :N] + acc_ref[1, :N]) / den[:, None] + bias_ref[...]
    y = _gn_elu(z, ng_ref[...], nb_ref[...], g8)
    y = jnp.dot(y, lin2w_ref[...].T, preferred_element_type=jnp.float32) + lin2b_ref[...]
    out_ref[...] = x_ref[...] + y


def _vspec():
    return pl.BlockSpec(memory_space=pltpu.VMEM)


_N_OUT4 = [
    jax.ShapeDtypeStruct((N, H), jnp.float32),
    jax.ShapeDtypeStruct((N, 1), jnp.float32),
    jax.ShapeDtypeStruct((N, 1), jnp.float32),
    jax.ShapeDtypeStruct((N, 1), jnp.float32),
]


def _dense_pre(x, p):
    return pl.pallas_call(
        _dense_pre_body, out_shape=_N_OUT4,
        in_specs=[_vspec()] * 10, out_specs=[_vspec()] * 4,
    )(x, p['lin1_W'], p['lin1_b'], p['conv1_W'], p['conv1_att_src'],
      p['conv1_att_dst'], p['pre_norm_g'], p['pre_norm_b'],
      p['norm1_g'], p['norm1_b'])


def _dense_mid(acc, den, p):
    return pl.pallas_call(
        _dense_mid_body, out_shape=_N_OUT4,
        in_specs=[_vspec()] * 8, out_specs=[_vspec()] * 4,
    )(acc, den, p['conv1_bias'], p['conv2_W'], p['conv2_att_src'],
      p['conv2_att_dst'], p['norm2_g'], p['norm2_b'])


def _dense_post(x, acc, den, p):
    return pl.pallas_call(
        _dense_post_body,
        out_shape=jax.ShapeDtypeStruct((N, C_OUT), jnp.float32),
        in_specs=[_vspec()] * 8, out_specs=_vspec(),
    )(x, acc, den, p['conv2_bias'], p['norm3_g'], p['norm3_b'],
      p['lin2_W'], p['lin2_b'])


def _sc_gat_body(h_hbm, as_hbm, ad_hbm, m_hbm, src_hbm, dst_hbm,
                 acc_out, den_out,
                 asb, adb, mb, srcb0, dstb0, eeb0, rows0,
                 srcb1, dstb1, eeb1, rows1, idxb,
                 dout, accs, dens, sem0, sem1):
    c = lax.axis_index("c")
    s = lax.axis_index("s")
    w = c * 16 + s
    zero16 = jnp.zeros((16,), jnp.float32)
    srcb = [srcb0, srcb1]
    dstb = [dstb0, dstb1]
    eeb = [eeb0, eeb1]
    rows = [rows0, rows1]
    sems = [sem0, sem1]

    pltpu.sync_copy(as_hbm, asb)
    pltpu.sync_copy(ad_hbm, adb)
    pltpu.sync_copy(m_hbm, mb)

    for r in range(80):
        for cc in range(4):
            rows0[r, pl.ds(cc * 16, 16)] = zero16
    def _zero_acc(j, _):
        pltpu.sync_copy(rows0.at[pl.ds(0, 80)],
                        accs.at[pl.ds(s * 640 + j * 80, 80)])
        return 0
    lax.fori_loop(0, 8, _zero_acc, 0)
    def _zero_dout(j, _):
        dout[pl.ds(j * 16, 16)] = zero16
        return 0
    lax.fori_loop(0, 40, _zero_dout, 0)
    pltpu.sync_copy(dout, dens.at[pl.ds(s * 640, 640)])
    plsc.subcore_barrier()

    def _scale_rows(rows_b, eeb_b, ngroups):
        def _sk(g, _):
            v = eeb_b[pl.ds(g * 16, 16)]
            for j in range(16):
                k = g * 16 + j
                bc = v[j]
                for cc in range(4):
                    sl = pl.ds(cc * 16, 16)
                    rows_b[k, sl] = rows_b[k, sl] * bc
            return 0
        lax.fori_loop(0, ngroups, _sk, 0)

    def _fire(b, ci):
        base = w * EPW + ci * CH
        pltpu.sync_copy(src_hbm.at[pl.ds(base, CH)], srcb[b])
        pltpu.sync_copy(dst_hbm.at[pl.ds(base, CH)], dstb[b])
        pltpu.async_copy(h_hbm.at[srcb[b]], rows[b], sems[b])

    def _proc(b):
        for g in range(CH // 16):
            sl = pl.ds(g * 16, 16)
            s16 = srcb[b][sl]
            d16 = dstb[b][sl]
            t = plsc.load_gather(asb, [s16]) + plsc.load_gather(adb, [d16])
            e = jnp.maximum(t, 0.2 * t)
            ee = jnp.exp(e - plsc.load_gather(mb, [d16]))
            eeb[b][sl] = ee
        pltpu.make_async_copy(h_hbm.at[srcb[b]], rows[b], sems[b]).wait()
        _scale_rows(rows[b], eeb[b], CH // 16)
        pltpu.sync_copy(rows[b], accs.at[dstb[b]], add=True)
        pltpu.sync_copy(eeb[b], dens.at[dstb[b]], add=True)

    _fire(0, 0)
    _fire(1, 1)
    def _edge_pair(g2, _):
        for b in range(2):
            _proc(b)
            _fire(b, 2 * g2 + b + 2)
        return 0
    lax.fori_loop(0, ECH // 2 - 1, _edge_pair, 0)
    _proc(0)
    _proc(1)

    def _self_chunk(ci, _):
        nbase = w * 320 + ci * 80
        pltpu.sync_copy(h_hbm.at[pl.ds(nbase, 80)], rows0.at[pl.ds(0, 80)])
        for g in range(5):
            sl = pl.ds(g * 16, 16)
            off = nbase + g * 16
            nsl = pl.ds(off, 16)
            t = asb[nsl] + adb[nsl]
            e = jnp.maximum(t, 0.2 * t)
            ee = jnp.exp(e - mb[nsl])
            i16 = lax.iota(jnp.int32, 16) + off
            valid = i16 < N
            ee = jnp.where(valid, ee, 0.0)
            idx = jnp.where(valid, i16, N)
            eeb0[sl] = ee
            idxb[sl] = idx
        _scale_rows(rows0, eeb0, 5)
        pltpu.sync_copy(rows0.at[pl.ds(0, 80)], accs.at[idxb], add=True)
        pltpu.sync_copy(eeb0.at[pl.ds(0, 80)], dens.at[idxb], add=True)
        return 0
    lax.fori_loop(0, 4, _self_chunk, 0)
    plsc.subcore_barrier()

    pltpu.sync_copy(accs.at[pl.ds(s * 640, 640)],
                    acc_out.at[c].at[pl.ds(s * 640, 640)])
    pltpu.sync_copy(dens.at[pl.ds(s * 640, 640)],
                    den_out.at[c].at[pl.ds(s * 640, 640)])


def _sc_gat(h_pad, a_s, a_d, m, src, dst):
    mesh = plsc.VectorSubcoreMesh(core_axis_name="c", subcore_axis_name="s")
    f32 = jnp.float32
    return pl.kernel(
        _sc_gat_body,
        out_type=[
            jax.ShapeDtypeStruct((2, NPAD, H), f32),
            jax.ShapeDtypeStruct((2, NPAD), f32),
        ],
        mesh=mesh,
        compiler_params=pltpu.CompilerParams(needs_layout_passes=False,
                                             use_tc_tiling_on_sc=False),
        scratch_types=[
            pltpu.VMEM((NPAD,), f32),
            pltpu.VMEM((NPAD,), f32),
            pltpu.VMEM((NPAD,), f32),
            pltpu.VMEM((CH,), jnp.int32),
            pltpu.VMEM((CH,), jnp.int32),
            pltpu.VMEM((CH,), f32),
            pltpu.VMEM((CH, H), f32),
            pltpu.VMEM((CH,), jnp.int32),
            pltpu.VMEM((CH,), jnp.int32),
            pltpu.VMEM((CH,), f32),
            pltpu.VMEM((CH, H), f32),
            pltpu.VMEM((80,), jnp.int32),
            pltpu.VMEM((640,), f32),
            pltpu.VMEM_SHARED((NPAD, H), f32),
            pltpu.VMEM_SHARED((NPAD,), f32),
            pltpu.SemaphoreType.DMA,
            pltpu.SemaphoreType.DMA,
        ],
    )(h_pad, a_s, a_d, m, src, dst)


def _pad_nodes(h, a_s, a_d, m):
    pad = NPAD - N
    return (jnp.pad(h, ((0, pad), (0, 0))),
            jnp.pad(a_s[:, 0], (0, pad)),
            jnp.pad(a_d[:, 0], (0, pad)),
            jnp.pad(m[:, 0], (0, pad)))


def kernel(x, edge_index, params):
    pad = jnp.full((EPAD - E,), N, edge_index.dtype)
    src = jnp.concatenate([edge_index[0], pad])
    dst = jnp.concatenate([edge_index[1], pad])
    h1, as1, ad1, m1 = _dense_pre(x, params)
    acc1, den1 = _sc_gat(*_pad_nodes(h1, as1, ad1, m1), src, dst)
    h2, as2, ad2, m2 = _dense_mid(acc1, den1, params)
    acc2, den2 = _sc_gat(*_pad_nodes(h2, as2, ad2, m2), src, dst)
    return _dense_post(x, acc2, den2, params)


# speedup vs baseline: 1.3769x; 1.3769x over previous
import jax
import jax.numpy as jnp
from jax import lax
from jax.experimental import pallas as pl
from jax.experimental.pallas import tpu as pltpu
from jax.experimental.pallas import tpu_sc as plsc

N = 10000
E = 320000
C_IN = 128
C_OUT = 128
H = 64
NPAD = 10240
CH = 80
ECH = 126
EPW = ECH * CH
EPAD = 32 * EPW


def _group_mats(c, groups):
    return jnp.repeat(jnp.eye(groups, dtype=jnp.float32), c // groups, axis=0)


def _gn_elu(x, gamma, beta, gmat, eps=1e-5):
    per = gmat.sum(axis=0)
    mean = ((x @ gmat) / per) @ gmat.T
    d = x - mean
    var = (((d * d) @ gmat) / per) @ gmat.T
    y = d * lax.rsqrt(var + eps) * gamma + beta
    return jnp.where(y > 0, y, jnp.exp(jnp.minimum(y, 0.0)) - 1.0)


def _att_outs(h, atts, attd, h_ref, as_ref, ad_ref, m_ref):
    a_s = jnp.sum(h * atts, axis=1, keepdims=True)
    a_d = jnp.sum(h * attd, axis=1, keepdims=True)
    t = jnp.max(a_s) + a_d
    m = jnp.maximum(t, 0.2 * t)
    h_ref[...] = h
    as_ref[...] = a_s
    ad_ref[...] = a_d
    m_ref[...] = m


def _dense_pre_body(x_ref, lin1w_ref, lin1b_ref, convw_ref, atts_ref, attd_ref,
                    preg_ref, preb_ref, n1g_ref, n1b_ref,
                    h_ref, as_ref, ad_ref, m_ref):
    g16 = _group_mats(C_IN, C_IN // 8)
    g8 = _group_mats(H, H // 8)
    y = _gn_elu(x_ref[...], preg_ref[...], preb_ref[...], g16)
    y = jnp.dot(y, lin1w_ref[...].T, preferred_element_type=jnp.float32) + lin1b_ref[...]
    y = _gn_elu(y, n1g_ref[...], n1b_ref[...], g8)
    h = jnp.dot(y, convw_ref[...].T, preferred_element_type=jnp.float32)
    _att_outs(h, atts_ref[...], attd_ref[...], h_ref, as_ref, ad_ref, m_ref)


def _dense_mid_body(acc_ref, den_ref, bias_ref, convw_ref, atts_ref, attd_ref,
                    ng_ref, nb_ref, h_ref, as_ref, ad_ref, m_ref):
    g8 = _group_mats(H, H // 8)
    den = den_ref[0, :N] + den_ref[1, :N] + 1e-16
    z = (acc_ref[0, :N] + acc_ref[1, :N]) / den[:, None] + bias_ref[...]
    y = _gn_elu(z, ng_ref[...], nb_ref[...], g8)
    h = jnp.dot(y, convw_ref[...].T, preferred_element_type=jnp.float32)
    _att_outs(h, atts_ref[...], attd_ref[...], h_ref, as_ref, ad_ref, m_ref)


def _dense_post_body(x_ref, acc_ref, den_ref, bias_ref, ng_ref, nb_ref,
                     lin2w_ref, lin2b_ref, out_ref):
    g8 = _group_mats(H, H // 8)
    den = den_ref[0, :N] + den_ref[1, :N] + 1e-16
    z = (acc_ref[0, :N] + acc_ref[1, :N]) / den[:, None] + bias_ref[...]
    y = _gn_elu(z, ng_ref[...], nb_ref[...], g8)
    y = jnp.dot(y, lin2w_ref[...].T, preferred_element_type=jnp.float32) + lin2b_ref[...]
    out_ref[...] = x_ref[...] + y


def _vspec():
    return pl.BlockSpec(memory_space=pltpu.VMEM)


_N_OUT4 = [
    jax.ShapeDtypeStruct((N, H), jnp.float32),
    jax.ShapeDtypeStruct((N, 1), jnp.float32),
    jax.ShapeDtypeStruct((N, 1), jnp.float32),
    jax.ShapeDtypeStruct((N, 1), jnp.float32),
]


def _dense_pre(x, p):
    return pl.pallas_call(
        _dense_pre_body, out_shape=_N_OUT4,
        in_specs=[_vspec()] * 10, out_specs=[_vspec()] * 4,
    )(x, p['lin1_W'], p['lin1_b'], p['conv1_W'], p['conv1_att_src'],
      p['conv1_att_dst'], p['pre_norm_g'], p['pre_norm_b'],
      p['norm1_g'], p['norm1_b'])


def _dense_mid(acc, den, p):
    return pl.pallas_call(
        _dense_mid_body, out_shape=_N_OUT4,
        in_specs=[_vspec()] * 8, out_specs=[_vspec()] * 4,
    )(acc, den, p['conv1_bias'], p['conv2_W'], p['conv2_att_src'],
      p['conv2_att_dst'], p['norm2_g'], p['norm2_b'])


def _dense_post(x, acc, den, p):
    return pl.pallas_call(
        _dense_post_body,
        out_shape=jax.ShapeDtypeStruct((N, C_OUT), jnp.float32),
        in_specs=[_vspec()] * 8, out_specs=_vspec(),
    )(x, acc, den, p['conv2_bias'], p['norm3_g'], p['norm3_b'],
      p['lin2_W'], p['lin2_b'])


def _sc_gat_body(h_hbm, as_hbm, ad_hbm, m_hbm, src_hbm, dst_hbm,
                 acc_out, den_out,
                 asb, adb, mb, srcall, dstall, dstb0, eeb0, rows0,
                 dstb1, eeb1, rows1, idxb,
                 dout, accs, dens, sem0, sem1):
    c = lax.axis_index("c")
    s = lax.axis_index("s")
    w = c * 16 + s
    zero16 = jnp.zeros((16,), jnp.float32)
    dstb = [dstb0, dstb1]
    eeb = [eeb0, eeb1]
    rows = [rows0, rows1]
    sems = [sem0, sem1]

    pltpu.sync_copy(as_hbm, asb)
    pltpu.sync_copy(ad_hbm, adb)
    pltpu.sync_copy(m_hbm, mb)
    pltpu.sync_copy(src_hbm.at[pl.ds(w * EPW, EPW)], srcall)
    pltpu.sync_copy(dst_hbm.at[pl.ds(w * EPW, EPW)], dstall)

    for r in range(80):
        for cc in range(4):
            rows0[r, pl.ds(cc * 16, 16)] = zero16
    def _zero_acc(j, _):
        pltpu.sync_copy(rows0.at[pl.ds(0, 80)],
                        accs.at[pl.ds(s * 640 + j * 80, 80)])
        return 0
    lax.fori_loop(0, 8, _zero_acc, 0)
    def _zero_dout(j, _):
        dout[pl.ds(j * 16, 16)] = zero16
        return 0
    lax.fori_loop(0, 40, _zero_dout, 0)
    pltpu.sync_copy(dout, dens.at[pl.ds(s * 640, 640)])
    plsc.subcore_barrier()

    def _scale_rows(rows_b, eeb_b, ngroups):
        def _sk(g, _):
            v = eeb_b[pl.ds(g * 16, 16)]
            for j in range(16):
                k = g * 16 + j
                bc = v[j]
                for cc in range(4):
                    sl = pl.ds(cc * 16, 16)
                    rows_b[k, sl] = rows_b[k, sl] * bc
            return 0
        lax.fori_loop(0, ngroups, _sk, 0)

    def _fire(b, ci):
        pltpu.async_copy(h_hbm.at[srcall.at[pl.ds(ci * CH, CH)]],
                         rows[b], sems[b])

    def _proc(b, ci):
        for g in range(CH // 16):
            sl = pl.ds(g * 16, 16)
            esl = pl.ds(ci * CH + g * 16, 16)
            s16 = srcall[esl]
            d16 = dstall[esl]
            t = plsc.load_gather(asb, [s16]) + plsc.load_gather(adb, [d16])
            e = jnp.maximum(t, 0.2 * t)
            ee = jnp.exp(e - plsc.load_gather(mb, [d16]))
            eeb[b][sl] = ee
            dstb[b][sl] = d16
        pltpu.make_async_copy(h_hbm.at[srcall.at[pl.ds(ci * CH, CH)]],
                              rows[b], sems[b]).wait()
        _scale_rows(rows[b], eeb[b], CH // 16)
        pltpu.sync_copy(rows[b], accs.at[dstb[b]], add=True)
        pltpu.sync_copy(eeb[b], dens.at[dstb[b]], add=True)

    _fire(0, 0)
    _fire(1, 1)
    def _edge_pair(g2, _):
        for b in range(2):
            ci = 2 * g2 + b
            _proc(b, ci)
            _fire(b, ci + 2)
        return 0
    lax.fori_loop(0, ECH // 2 - 1, _edge_pair, 0)
    _proc(0, ECH - 2)
    _proc(1, ECH - 1)

    def _self_chunk(ci, _):
        nbase = w * 320 + ci * 80
        pltpu.sync_copy(h_hbm.at[pl.ds(nbase, 80)], rows0.at[pl.ds(0, 80)])
        for g in range(5):
            sl = pl.ds(g * 16, 16)
            off = nbase + g * 16
            nsl = pl.ds(off, 16)
            t = asb[nsl] + adb[nsl]
            e = jnp.maximum(t, 0.2 * t)
            ee = jnp.exp(e - mb[nsl])
            i16 = lax.iota(jnp.int32, 16) + off
            valid = i16 < N
            ee = jnp.where(valid, ee, 0.0)
            idx = jnp.where(valid, i16, N)
            eeb0[sl] = ee
            idxb[sl] = idx
        _scale_rows(rows0, eeb0, 5)
        pltpu.sync_copy(rows0.at[pl.ds(0, 80)], accs.at[idxb], add=True)
        pltpu.sync_copy(eeb0.at[pl.ds(0, 80)], dens.at[idxb], add=True)
        return 0
    lax.fori_loop(0, 4, _self_chunk, 0)
    plsc.subcore_barrier()

    pltpu.sync_copy(accs.at[pl.ds(s * 640, 640)],
                    acc_out.at[c].at[pl.ds(s * 640, 640)])
    pltpu.sync_copy(dens.at[pl.ds(s * 640, 640)],
                    den_out.at[c].at[pl.ds(s * 640, 640)])


def _sc_gat(h_pad, a_s, a_d, m, src, dst):
    mesh = plsc.VectorSubcoreMesh(core_axis_name="c", subcore_axis_name="s")
    f32 = jnp.float32
    return pl.kernel(
        _sc_gat_body,
        out_type=[
            jax.ShapeDtypeStruct((2, NPAD, H), f32),
            jax.ShapeDtypeStruct((2, NPAD), f32),
        ],
        mesh=mesh,
        compiler_params=pltpu.CompilerParams(needs_layout_passes=False,
                                             use_tc_tiling_on_sc=False),
        scratch_types=[
            pltpu.VMEM((NPAD,), f32),
            pltpu.VMEM((NPAD,), f32),
            pltpu.VMEM((NPAD,), f32),
            pltpu.VMEM((EPW,), jnp.int32),
            pltpu.VMEM((EPW,), jnp.int32),
            pltpu.VMEM((CH,), jnp.int32),
            pltpu.VMEM((CH,), f32),
            pltpu.VMEM((CH, H), f32),
            pltpu.VMEM((CH,), jnp.int32),
            pltpu.VMEM((CH,), f32),
            pltpu.VMEM((CH, H), f32),
            pltpu.VMEM((80,), jnp.int32),
            pltpu.VMEM((640,), f32),
            pltpu.VMEM_SHARED((NPAD, H), f32),
            pltpu.VMEM_SHARED((NPAD,), f32),
            pltpu.SemaphoreType.DMA,
            pltpu.SemaphoreType.DMA,
        ],
    )(h_pad, a_s, a_d, m, src, dst)


def _pad_nodes(h, a_s, a_d, m):
    pad = NPAD - N
    return (jnp.pad(h, ((0, pad), (0, 0))),
            jnp.pad(a_s[:, 0], (0, pad)),
            jnp.pad(a_d[:, 0], (0, pad)),
            jnp.pad(m[:, 0], (0, pad)))


def kernel(x, edge_index, params):
    pad = jnp.full((EPAD - E,), N, edge_index.dtype)
    src = jnp.concatenate([edge_index[0], pad])
    dst = jnp.concatenate([edge_index[1], pad])
    h1, as1, ad1, m1 = _dense_pre(x, params)
    acc1, den1 = _sc_gat(*_pad_nodes(h1, as1, ad1, m1), src, dst)
    h2, as2, ad2, m2 = _dense_mid(acc1, den1, params)
    acc2, den2 = _sc_gat(*_pad_nodes(h2, as2, ad2, m2), src, dst)
    return _dense_post(x, acc2, den2, params)


# speedup vs baseline: 1.3890x; 1.0088x over previous
import jax
import jax.numpy as jnp
from jax import lax
from jax.experimental import pallas as pl
from jax.experimental.pallas import tpu as pltpu
from jax.experimental.pallas import tpu_sc as plsc

N = 10000
E = 320000
C_IN = 128
C_OUT = 128
H = 64
NPAD = 10240
CH = 80
ECH = 126
EPW = ECH * CH
EPAD = 32 * EPW


def _group_mats(c, groups):
    return jnp.repeat(jnp.eye(groups, dtype=jnp.float32), c // groups, axis=0)


def _gn_elu(x, gamma, beta, gmat, eps=1e-5):
    per = gmat.sum(axis=0)
    mean = ((x @ gmat) / per) @ gmat.T
    d = x - mean
    var = (((d * d) @ gmat) / per) @ gmat.T
    y = d * lax.rsqrt(var + eps) * gamma + beta
    return jnp.where(y > 0, y, jnp.exp(jnp.minimum(y, 0.0)) - 1.0)


def _att_outs(h, atts, attd, h_ref, as_ref, ad_ref, m_ref):
    a_s = jnp.sum(h * atts, axis=1, keepdims=True)
    a_d = jnp.sum(h * attd, axis=1, keepdims=True)
    t = jnp.max(a_s) + a_d
    m = jnp.maximum(t, 0.2 * t)
    h_ref[...] = h
    as_ref[...] = a_s
    ad_ref[...] = a_d
    m_ref[...] = m


def _dense_pre_body(x_ref, lin1w_ref, lin1b_ref, convw_ref, atts_ref, attd_ref,
                    preg_ref, preb_ref, n1g_ref, n1b_ref,
                    h_ref, as_ref, ad_ref, m_ref):
    g16 = _group_mats(C_IN, C_IN // 8)
    g8 = _group_mats(H, H // 8)
    y = _gn_elu(x_ref[...], preg_ref[...], preb_ref[...], g16)
    y = jnp.dot(y, lin1w_ref[...].T, preferred_element_type=jnp.float32) + lin1b_ref[...]
    y = _gn_elu(y, n1g_ref[...], n1b_ref[...], g8)
    h = jnp.dot(y, convw_ref[...].T, preferred_element_type=jnp.float32)
    _att_outs(h, atts_ref[...], attd_ref[...], h_ref, as_ref, ad_ref, m_ref)


def _dense_mid_body(acc_ref, den_ref, bias_ref, convw_ref, atts_ref, attd_ref,
                    ng_ref, nb_ref, h_ref, as_ref, ad_ref, m_ref):
    g8 = _group_mats(H, H // 8)
    den = den_ref[0, :N] + den_ref[1, :N] + 1e-16
    z = (acc_ref[0, :N] + acc_ref[1, :N]) / den[:, None] + bias_ref[...]
    y = _gn_elu(z, ng_ref[...], nb_ref[...], g8)
    h = jnp.dot(y, convw_ref[...].T, preferred_element_type=jnp.float32)
    _att_outs(h, atts_ref[...], attd_ref[...], h_ref, as_ref, ad_ref, m_ref)


def _dense_post_body(x_ref, acc_ref, den_ref, bias_ref, ng_ref, nb_ref,
                     lin2w_ref, lin2b_ref, out_ref):
    g8 = _group_mats(H, H // 8)
    den = den_ref[0, :N] + den_ref[1, :N] + 1e-16
    z = (acc_ref[0, :N] + acc_ref[1, :N]) / den[:, None] + bias_ref[...]
    y = _gn_elu(z, ng_ref[...], nb_ref[...], g8)
    y = jnp.dot(y, lin2w_ref[...].T, preferred_element_type=jnp.float32) + lin2b_ref[...]
    out_ref[...] = x_ref[...] + y


def _vspec():
    return pl.BlockSpec(memory_space=pltpu.VMEM)


_N_OUT4 = [
    jax.ShapeDtypeStruct((N, H), jnp.float32),
    jax.ShapeDtypeStruct((N, 1), jnp.float32),
    jax.ShapeDtypeStruct((N, 1), jnp.float32),
    jax.ShapeDtypeStruct((N, 1), jnp.float32),
]


def _dense_pre(x, p):
    return pl.pallas_call(
        _dense_pre_body, out_shape=_N_OUT4,
        in_specs=[_vspec()] * 10, out_specs=[_vspec()] * 4,
    )(x, p['lin1_W'], p['lin1_b'], p['conv1_W'], p['conv1_att_src'],
      p['conv1_att_dst'], p['pre_norm_g'], p['pre_norm_b'],
      p['norm1_g'], p['norm1_b'])


def _dense_mid(acc, den, p):
    return pl.pallas_call(
        _dense_mid_body, out_shape=_N_OUT4,
        in_specs=[_vspec()] * 8, out_specs=[_vspec()] * 4,
    )(acc, den, p['conv1_bias'], p['conv2_W'], p['conv2_att_src'],
      p['conv2_att_dst'], p['norm2_g'], p['norm2_b'])


def _dense_post(x, acc, den, p):
    return pl.pallas_call(
        _dense_post_body,
        out_shape=jax.ShapeDtypeStruct((N, C_OUT), jnp.float32),
        in_specs=[_vspec()] * 8, out_specs=_vspec(),
    )(x, acc, den, p['conv2_bias'], p['norm3_g'], p['norm3_b'],
      p['lin2_W'], p['lin2_b'])


def _sc_gat_body(h_hbm, as_hbm, ad_hbm, m_hbm, src_hbm, dst_hbm,
                 acc_out, den_out,
                 asb, adb, mb, srcall, dstall, dstb0, eeb0, rows0,
                 dstb1, eeb1, rows1, dstb2, eeb2, rows2, idxb,
                 dout, accs, dens, sem0, sem1, sem2):
    c = lax.axis_index("c")
    s = lax.axis_index("s")
    w = c * 16 + s
    zero16 = jnp.zeros((16,), jnp.float32)
    dstb = [dstb0, dstb1, dstb2]
    eeb = [eeb0, eeb1, eeb2]
    rows = [rows0, rows1, rows2]
    sems = [sem0, sem1, sem2]

    pltpu.sync_copy(as_hbm, asb)
    pltpu.sync_copy(ad_hbm, adb)
    pltpu.sync_copy(m_hbm, mb)
    pltpu.sync_copy(src_hbm.at[pl.ds(w * EPW, EPW)], srcall)
    pltpu.sync_copy(dst_hbm.at[pl.ds(w * EPW, EPW)], dstall)

    for r in range(80):
        for cc in range(4):
            rows0[r, pl.ds(cc * 16, 16)] = zero16
    def _zero_acc(j, _):
        pltpu.sync_copy(rows0.at[pl.ds(0, 80)],
                        accs.at[pl.ds(s * 640 + j * 80, 80)])
        return 0
    lax.fori_loop(0, 8, _zero_acc, 0)
    def _zero_dout(j, _):
        dout[pl.ds(j * 16, 16)] = zero16
        return 0
    lax.fori_loop(0, 40, _zero_dout, 0)
    pltpu.sync_copy(dout, dens.at[pl.ds(s * 640, 640)])
    plsc.subcore_barrier()

    def _scale_rows(rows_b, eeb_b, ngroups):
        def _sk(g, _):
            v = eeb_b[pl.ds(g * 16, 16)]
            for j in range(16):
                k = g * 16 + j
                bc = v[j]
                for cc in range(4):
                    sl = pl.ds(cc * 16, 16)
                    rows_b[k, sl] = rows_b[k, sl] * bc
            return 0
        lax.fori_loop(0, ngroups, _sk, 0)

    def _fire(b, ci):
        pltpu.async_copy(h_hbm.at[srcall.at[pl.ds(ci * CH, CH)]],
                         rows[b], sems[b])

    def _proc(b, ci):
        for g in range(CH // 16):
            sl = pl.ds(g * 16, 16)
            esl = pl.ds(ci * CH + g * 16, 16)
            s16 = srcall[esl]
            d16 = dstall[esl]
            t = plsc.load_gather(asb, [s16]) + plsc.load_gather(adb, [d16])
            e = jnp.maximum(t, 0.2 * t)
            ee = jnp.exp(e - plsc.load_gather(mb, [d16]))
            eeb[b][sl] = ee
            dstb[b][sl] = d16
        pltpu.make_async_copy(h_hbm.at[srcall.at[pl.ds(ci * CH, CH)]],
                              rows[b], sems[b]).wait()
        _scale_rows(rows[b], eeb[b], CH // 16)
        pltpu.sync_copy(rows[b], accs.at[dstb[b]], add=True)
        pltpu.sync_copy(eeb[b], dens.at[dstb[b]], add=True)

    for b in range(3):
        _fire(b, b)
    def _edge_trio(g3, _):
        for b in range(3):
            ci = 3 * g3 + b
            _proc(b, ci)
            _fire(b, ci + 3)
        return 0
    lax.fori_loop(0, ECH // 3 - 1, _edge_trio, 0)
    for b in range(3):
        _proc(b, ECH - 3 + b)

    def _self_chunk(ci, _):
        nbase = w * 320 + ci * 80
        pltpu.sync_copy(h_hbm.at[pl.ds(nbase, 80)], rows0.at[pl.ds(0, 80)])
        for g in range(5):
            sl = pl.ds(g * 16, 16)
            off = nbase + g * 16
            nsl = pl.ds(off, 16)
            t = asb[nsl] + adb[nsl]
            e = jnp.maximum(t, 0.2 * t)
            ee = jnp.exp(e - mb[nsl])
            i16 = lax.iota(jnp.int32, 16) + off
            valid = i16 < N
            ee = jnp.where(valid, ee, 0.0)
            idx = jnp.where(valid, i16, N)
            eeb0[sl] = ee
            idxb[sl] = idx
        _scale_rows(rows0, eeb0, 5)
        pltpu.sync_copy(rows0.at[pl.ds(0, 80)], accs.at[idxb], add=True)
        pltpu.sync_copy(eeb0.at[pl.ds(0, 80)], dens.at[idxb], add=True)
        return 0
    lax.fori_loop(0, 4, _self_chunk, 0)
    plsc.subcore_barrier()

    pltpu.sync_copy(accs.at[pl.ds(s * 640, 640)],
                    acc_out.at[c].at[pl.ds(s * 640, 640)])
    pltpu.sync_copy(dens.at[pl.ds(s * 640, 640)],
                    den_out.at[c].at[pl.ds(s * 640, 640)])


def _sc_gat(h_pad, a_s, a_d, m, src, dst):
    mesh = plsc.VectorSubcoreMesh(core_axis_name="c", subcore_axis_name="s")
    f32 = jnp.float32
    return pl.kernel(
        _sc_gat_body,
        out_type=[
            jax.ShapeDtypeStruct((2, NPAD, H), f32),
            jax.ShapeDtypeStruct((2, NPAD), f32),
        ],
        mesh=mesh,
        compiler_params=pltpu.CompilerParams(needs_layout_passes=False,
                                             use_tc_tiling_on_sc=False),
        scratch_types=[
            pltpu.VMEM((NPAD,), f32),
            pltpu.VMEM((NPAD,), f32),
            pltpu.VMEM((NPAD,), f32),
            pltpu.VMEM((EPW,), jnp.int32),
            pltpu.VMEM((EPW,), jnp.int32),
            pltpu.VMEM((CH,), jnp.int32),
            pltpu.VMEM((CH,), f32),
            pltpu.VMEM((CH, H), f32),
            pltpu.VMEM((CH,), jnp.int32),
            pltpu.VMEM((CH,), f32),
            pltpu.VMEM((CH, H), f32),
            pltpu.VMEM((CH,), jnp.int32),
            pltpu.VMEM((CH,), f32),
            pltpu.VMEM((CH, H), f32),
            pltpu.VMEM((80,), jnp.int32),
            pltpu.VMEM((640,), f32),
            pltpu.VMEM_SHARED((NPAD, H), f32),
            pltpu.VMEM_SHARED((NPAD,), f32),
            pltpu.SemaphoreType.DMA,
            pltpu.SemaphoreType.DMA,
            pltpu.SemaphoreType.DMA,
        ],
    )(h_pad, a_s, a_d, m, src, dst)


def _pad_nodes(h, a_s, a_d, m):
    pad = NPAD - N
    return (jnp.pad(h, ((0, pad), (0, 0))),
            jnp.pad(a_s[:, 0], (0, pad)),
            jnp.pad(a_d[:, 0], (0, pad)),
            jnp.pad(m[:, 0], (0, pad)))


def kernel(x, edge_index, params):
    pad = jnp.full((EPAD - E,), N, edge_index.dtype)
    src = jnp.concatenate([edge_index[0], pad])
    dst = jnp.concatenate([edge_index[1], pad])
    h1, as1, ad1, m1 = _dense_pre(x, params)
    acc1, den1 = _sc_gat(*_pad_nodes(h1, as1, ad1, m1), src, dst)
    h2, as2, ad2, m2 = _dense_mid(acc1, den1, params)
    acc2, den2 = _sc_gat(*_pad_nodes(h2, as2, ad2, m2), src, dst)
    return _dense_post(x, acc2, den2, params)
